# baseline (speedup 1.0000x reference)
import functools

import jax
import jax.numpy as jnp
from jax import lax
from jax.experimental import pallas as pl
from jax.experimental.pallas import tpu as pltpu
from jax.experimental.pallas import tpu_sc as plsc

_NC = 2
_NS = 16
_NW = _NC * _NS

_B = 1024 * 77
_D = 768
_BPW = _B // _NW
_CH = 32
_NBUF = 4


def _make_gather(ch, nbuf):
    nchunk = _BPW // ch
    npair = -(-nchunk // nbuf)
    mesh = plsc.VectorSubcoreMesh(
        core_axis_name="c", subcore_axis_name="s",
        num_cores=_NC, num_subcores=_NS)

    scratch = [
        pltpu.VMEM((_BPW,), jnp.int32),
        pltpu.VMEM((nbuf, ch, _D), jnp.float32),
    ] + [pltpu.SemaphoreType.DMA] * (2 * nbuf)

    @functools.partial(
        pl.kernel,
        mesh=mesh,
        out_type=jax.ShapeDtypeStruct((_B, _D), jnp.float32),
        scratch_types=scratch,
    )
    def gather_kernel(idx_hbm, table_hbm, out_hbm, idx_v, rows_v, *sems):
        gsems = sems[:nbuf]
        osems = sems[nbuf:]
        wid = lax.axis_index("s") * _NC + lax.axis_index("c")
        base = wid * _BPW
        pltpu.sync_copy(idx_hbm.at[pl.ds(base, _BPW)], idx_v)

        def g_start(c, slot):
            pltpu.async_copy(table_hbm.at[idx_v.at[pl.ds(c * ch, ch)]],
                             rows_v.at[slot], gsems[slot])

        def g_wait(c, slot):
            pltpu.make_async_copy(table_hbm.at[idx_v.at[pl.ds(c * ch, ch)]],
                                  rows_v.at[slot], gsems[slot]).wait()

        def o_start(c, slot):
            pltpu.async_copy(rows_v.at[slot],
                             out_hbm.at[pl.ds(base + c * ch, ch)],
                             osems[slot])

        def o_wait(c, slot):
            pltpu.make_async_copy(rows_v.at[slot],
                                  out_hbm.at[pl.ds(base + c * ch, ch)],
                                  osems[slot]).wait()

        for c in range(min(nbuf, nchunk)):
            g_start(c, c)

        def body(p, _):
            for slot in range(nbuf):
                c = p * nbuf + slot

                @pl.when(c < nchunk)
                def _():
                    g_wait(c, slot)
                    o_start(c, slot)

                prev = c - 1
                pslot = (slot - 1) % nbuf

                @pl.when((prev >= 0) & (prev + nbuf < nchunk))
                def _():
                    o_wait(prev, pslot)
                    g_start(prev + nbuf, pslot)
            return 0

        lax.fori_loop(0, npair, body, 0)
        for c in range(max(0, nchunk - nbuf), nchunk):
            o_wait(c, c % nbuf)

    return gather_kernel


_gather = _make_gather(_CH, _NBUF)


@jax.jit
def kernel(input_ids, table):
    ids_flat = input_ids.reshape(_B)
    out = _gather(ids_flat, table)
    return out.reshape(input_ids.shape + (table.shape[1],))


# speedup vs baseline: 1.5525x; 1.5525x over previous
import functools

import jax
import jax.numpy as jnp
from jax import lax
from jax.experimental import pallas as pl
from jax.experimental.pallas import tpu as pltpu
from jax.experimental.pallas import tpu_sc as plsc

_NC = 2
_NS = 16
_NW = _NC * _NS

_NB = 1024
_S = 77
_D = 768
_BPW = _NB // _NW


def _make_gather():
    mesh = plsc.VectorSubcoreMesh(
        core_axis_name="c", subcore_axis_name="s",
        num_cores=_NC, num_subcores=_NS)

    @functools.partial(
        pl.kernel,
        mesh=mesh,
        out_type=jax.ShapeDtypeStruct((_NB, _S, _D), jnp.float32),
        scratch_types=[
            pltpu.VMEM((_BPW, _S), jnp.int32),
            pltpu.VMEM((2, _S, _D), jnp.float32),
            pltpu.SemaphoreType.DMA,
            pltpu.SemaphoreType.DMA,
            pltpu.SemaphoreType.DMA,
            pltpu.SemaphoreType.DMA,
        ],
    )
    def gather_kernel(idx_hbm, table_hbm, out_hbm, idx_v, rows_v,
                      gsem0, gsem1, osem0, osem1):
        wid = lax.axis_index("s") * _NC + lax.axis_index("c")
        base = wid * _BPW
        pltpu.sync_copy(idx_hbm.at[pl.ds(base, _BPW)], idx_v)

        gsems = (gsem0, gsem1)
        osems = (osem0, osem1)

        def g_start(j, slot):
            pltpu.async_copy(table_hbm.at[idx_v.at[j]], rows_v.at[slot],
                             gsems[slot])

        def g_wait(j, slot):
            pltpu.make_async_copy(table_hbm.at[idx_v.at[j]], rows_v.at[slot],
                                  gsems[slot]).wait()

        def o_start(j, slot):
            pltpu.async_copy(rows_v.at[slot], out_hbm.at[base + j],
                             osems[slot])

        def o_wait(j, slot):
            pltpu.make_async_copy(rows_v.at[slot], out_hbm.at[base + j],
                                  osems[slot]).wait()

        g_start(0, 0)
        g_start(1, 1)

        def body(p, _):
            for slot in range(2):
                j = 2 * p + slot
                g_wait(j, slot)
                o_start(j, slot)

                @pl.when(j + 2 < _BPW)
                def _():
                    o_wait(j, slot)
                    g_start(j + 2, slot)
            return 0

        lax.fori_loop(0, _BPW // 2, body, 0)
        for slot, j in ((0, _BPW - 2), (1, _BPW - 1)):
            o_wait(j, slot)

    return gather_kernel


_gather = _make_gather()


@jax.jit
def kernel(input_ids, table):
    return _gather(input_ids, table)
